# Initial kernel scaffold; baseline (speedup 1.0000x reference)
import jax
import jax.numpy as jnp
from jax.experimental import pallas as pl


def kernel(query, memory_keys, memory_values, memory_importance):
    raise NotImplementedError("write your pallas kernel here")


# speedup vs baseline: 1.9646x; 1.9646x over previous
import functools

import jax
import jax.numpy as jnp
from jax.experimental import pallas as pl
from jax.experimental.pallas import tpu as pltpu

Q = 1024
K = 16384
D = 768
TOPK = 8
NEG = -3.0e38

BQ = 256
BK = 2048
NQ = Q // BQ
NK = K // BK


def _extract_topk(s, gidx, n):
    r, w = s.shape
    cols = jax.lax.broadcasted_iota(jnp.int32, (r, w), 1)
    vals, idxs = [], []
    for _ in range(n):
        m = jnp.max(s, axis=1, keepdims=True)
        pos = jnp.min(jnp.where(s >= m, cols, w), axis=1, keepdims=True)
        hit = cols == pos
        sel = jnp.min(jnp.where(hit, gidx, jnp.int32(2**31 - 1)), axis=1,
                      keepdims=True)
        s = jnp.where(hit, NEG, s)
        vals.append(m)
        idxs.append(sel)
    return jnp.concatenate(vals, axis=1), jnp.concatenate(idxs, axis=1)


def _topk_body(q_ref, k_ref, imp_ref, attn_ref, idx_ref, qn_s, rv_s, ri_s):
    ki = pl.program_id(1)

    @pl.when(ki == 0)
    def _():
        q = q_ref[...]
        nrm = jnp.maximum(jnp.sqrt(jnp.sum(q * q, axis=1, keepdims=True)),
                          1e-8)
        qn_s[...] = q / nrm

    kb = k_ref[...]
    knrm = jnp.maximum(jnp.sqrt(jnp.sum(kb * kb, axis=1, keepdims=True)),
                       1e-8)
    kn = kb / knrm
    s = jax.lax.dot_general(qn_s[...], kn, (((1,), (1,)), ((), ())),
                            preferred_element_type=jnp.float32)
    s = s * 0.7 + imp_ref[0, :][None, :] * 0.3
    gidx = (jax.lax.broadcasted_iota(jnp.int32, (BQ, BK), 1) + ki * BK)
    cv, ci = _extract_topk(s, gidx, TOPK)

    @pl.when(ki == 0)
    def _():
        rv_s[...] = cv
        ri_s[...] = ci

    @pl.when(ki > 0)
    def _():
        mv = jnp.concatenate([rv_s[...], cv], axis=1)
        mi = jnp.concatenate([ri_s[...], ci], axis=1)
        nv, ni = _extract_topk(mv, mi, TOPK)
        rv_s[...] = nv
        ri_s[...] = ni

    @pl.when(ki == NK - 1)
    def _():
        tv = rv_s[...]
        e = jnp.exp(tv - jnp.max(tv, axis=1, keepdims=True))
        attn_ref[...] = e / jnp.sum(e, axis=1, keepdims=True)
        idx_ref[...] = ri_s[...]


def _topk_call(query, memory_keys, imp2d):
    return pl.pallas_call(
        _topk_body,
        grid=(NQ, NK),
        in_specs=[
            pl.BlockSpec((BQ, D), lambda qi, ki: (qi, 0)),
            pl.BlockSpec((BK, D), lambda qi, ki: (ki, 0)),
            pl.BlockSpec((1, BK), lambda qi, ki: (0, ki)),
        ],
        out_specs=[
            pl.BlockSpec((BQ, TOPK), lambda qi, ki: (qi, 0)),
            pl.BlockSpec((BQ, TOPK), lambda qi, ki: (qi, 0)),
        ],
        out_shape=[
            jax.ShapeDtypeStruct((Q, TOPK), jnp.float32),
            jax.ShapeDtypeStruct((Q, TOPK), jnp.int32),
        ],
        scratch_shapes=[
            pltpu.VMEM((BQ, D), jnp.float32),
            pltpu.VMEM((BQ, TOPK), jnp.float32),
            pltpu.VMEM((BQ, TOPK), jnp.int32),
        ],
        compiler_params=pltpu.CompilerParams(
            dimension_semantics=("parallel", "arbitrary")),
    )(query, memory_keys, imp2d)


def _fuse_body(attn_ref, idx_ref, q_ref, v_ref, out_ref, acc_s):
    ki = pl.program_id(1)
    idx = idx_ref[...]
    w = attn_ref[...] * 0.08
    cols = (jax.lax.broadcasted_iota(jnp.int32, (BQ, BK), 1) + ki * BK)
    wmat = jnp.zeros((BQ, BK), jnp.float32)
    for j in range(TOPK):
        wmat = wmat + jnp.where(cols == idx[:, j:j + 1], w[:, j:j + 1], 0.0)
    part = jax.lax.dot_general(
        wmat.astype(jnp.bfloat16), v_ref[...].astype(jnp.bfloat16),
        (((1,), (0,)), ((), ())), preferred_element_type=jnp.float32)

    @pl.when(ki == 0)
    def _():
        acc_s[...] = q_ref[...]

    acc_s[...] += part

    @pl.when(ki == NK - 1)
    def _():
        out_ref[...] = acc_s[...]


def _fuse_call(attn, idx, query, memory_values):
    return pl.pallas_call(
        _fuse_body,
        grid=(NQ, NK),
        in_specs=[
            pl.BlockSpec((BQ, TOPK), lambda qi, ki: (qi, 0)),
            pl.BlockSpec((BQ, TOPK), lambda qi, ki: (qi, 0)),
            pl.BlockSpec((BQ, D), lambda qi, ki: (qi, 0)),
            pl.BlockSpec((BK, D), lambda qi, ki: (ki, 0)),
        ],
        out_specs=pl.BlockSpec((BQ, D), lambda qi, ki: (qi, 0)),
        out_shape=jax.ShapeDtypeStruct((Q, D), jnp.float32),
        scratch_shapes=[pltpu.VMEM((BQ, D), jnp.float32)],
        compiler_params=pltpu.CompilerParams(
            dimension_semantics=("parallel", "arbitrary")),
    )(attn, idx, query, memory_values)


def kernel(query, memory_keys, memory_values, memory_importance):
    imp2d = memory_importance.reshape(1, K)
    attn, idx = _topk_call(query, memory_keys, imp2d)
    return _fuse_call(attn, idx, query, memory_values)


# speedup vs baseline: 5.4475x; 2.7728x over previous
import functools

import jax
import jax.numpy as jnp
from jax import lax
from jax.experimental import pallas as pl
from jax.experimental.pallas import tpu as pltpu
from jax.experimental.pallas import tpu_sc as plsc

Q = 1024
K = 16384
D = 768
TOPK = 8
NEG = -3.0e38

BQ = 512
BK = 4096
NQ = Q // BQ
NK = K // BK


INTMIN = -(2**31)
LOWMASK = 2**12 - 1
HIMASK = -(2**12)
NC_CH = TOPK + 4 * 4


def _extract_keys(key, n):
    ms = []
    cur = key
    for j in range(n):
        if j > 0:
            cur = jnp.where(cur < ms[-1], cur, INTMIN)
        ms.append(jnp.max(cur, axis=1, keepdims=True))
    return ms


def _topk_body(q_ref, k_ref, imp_ref, attn_ref, idx_ref, qn_s, rk_s):
    ki = pl.program_id(1)

    @pl.when(ki == 0)
    def _():
        q = q_ref[...]
        nrm = jnp.maximum(jnp.sqrt(jnp.sum(q * q, axis=1, keepdims=True)),
                          1e-8)
        qn_s[...] = q / nrm

    kb = k_ref[...]
    knrm = jnp.maximum(jnp.sqrt(jnp.sum(kb * kb, axis=1, keepdims=True)),
                       1e-8)
    kn = kb / knrm
    s = jax.lax.dot_general(qn_s[...], kn, (((1,), (1,)), ((), ())),
                            preferred_element_type=jnp.float32)
    s = s * 0.7 + imp_ref[0, :][None, :] * 0.3 + 1.0
    b = jax.lax.bitcast_convert_type(s, jnp.int32)
    revcols = (BK - 1) - jax.lax.broadcasted_iota(jnp.int32, (BQ, BK), 1)
    key = (b & HIMASK) | revcols
    half = BK // 2
    a1 = jnp.maximum(key[:, :half], key[:, half:])
    b1 = jnp.minimum(key[:, :half], key[:, half:])
    a2 = jnp.maximum(a1[:, :half // 2], a1[:, half // 2:])
    b2 = jnp.minimum(a1[:, :half // 2], a1[:, half // 2:])
    a3 = jnp.maximum(a2[:, :half // 4], a2[:, half // 4:])
    b3 = jnp.minimum(a2[:, :half // 4], a2[:, half // 4:])
    a4 = jnp.maximum(a3[:, :half // 8], a3[:, half // 8:])
    b4 = jnp.minimum(a3[:, :half // 8], a3[:, half // 8:])
    cands = (_extract_keys(a4, TOPK) + _extract_keys(b4, 4)
             + _extract_keys(b3, 4) + _extract_keys(b2, 4)
             + _extract_keys(b1, 4))
    rk_s[ki] = jnp.concatenate(cands, axis=1)

    @pl.when(ki == NK - 1)
    def _():
        mk = rk_s[...]
        chunk3 = jax.lax.broadcasted_iota(jnp.int32, (NK, BQ, NC_CH), 0)
        gidx3 = chunk3 * BK + (BK - 1) - (mk & LOWMASK)
        ks, idxs = [], []
        for _ in range(TOPK):
            m = jnp.max(jnp.max(mk, axis=0, keepdims=True), axis=2,
                        keepdims=True)
            hit = mk == m
            sel = jnp.max(jnp.max(jnp.where(hit, gidx3, -1),
                                  axis=0, keepdims=True), axis=2,
                          keepdims=True)
            mk = jnp.where(hit, INTMIN, mk)
            ks.append(m)
            idxs.append(sel)
        tk = jnp.concatenate(ks, axis=2)[0]
        tidx = jnp.concatenate(idxs, axis=2)[0]
        tv = jax.lax.bitcast_convert_type(tk & HIMASK, jnp.float32)
        e = jnp.exp(tv - jnp.max(tv, axis=1, keepdims=True))
        attn_ref[...] = e / jnp.sum(e, axis=1, keepdims=True)
        idx_ref[...] = tidx


def _topk_call(query, memory_keys, imp2d):
    return pl.pallas_call(
        _topk_body,
        grid=(NQ, NK),
        in_specs=[
            pl.BlockSpec((BQ, D), lambda qi, ki: (qi, 0)),
            pl.BlockSpec((BK, D), lambda qi, ki: (ki, 0)),
            pl.BlockSpec((1, BK), lambda qi, ki: (0, ki)),
        ],
        out_specs=[
            pl.BlockSpec((BQ, TOPK), lambda qi, ki: (qi, 0)),
            pl.BlockSpec((BQ, TOPK), lambda qi, ki: (qi, 0)),
        ],
        out_shape=[
            jax.ShapeDtypeStruct((Q, TOPK), jnp.float32),
            jax.ShapeDtypeStruct((Q, TOPK), jnp.int32),
        ],
        scratch_shapes=[
            pltpu.VMEM((BQ, D), jnp.float32),
            pltpu.VMEM((NK, BQ, NC_CH), jnp.int32),
        ],
        compiler_params=pltpu.CompilerParams(
            dimension_semantics=("parallel", "arbitrary")),
    )(query, memory_keys, imp2d)


def _fuse_body(attn_ref, idx_ref, q_ref, v_ref, out_ref, acc_s):
    ki = pl.program_id(1)
    idx = idx_ref[...]
    w = attn_ref[...] * 0.08
    cols = (jax.lax.broadcasted_iota(jnp.int32, (BQ, BK), 1) + ki * BK)
    wmat = jnp.zeros((BQ, BK), jnp.float32)
    for j in range(TOPK):
        wmat = wmat + jnp.where(cols == idx[:, j:j + 1], w[:, j:j + 1], 0.0)
    part = jax.lax.dot_general(
        wmat.astype(jnp.bfloat16), v_ref[...].astype(jnp.bfloat16),
        (((1,), (0,)), ((), ())), preferred_element_type=jnp.float32)

    @pl.when(ki == 0)
    def _():
        acc_s[...] = q_ref[...]

    acc_s[...] += part

    @pl.when(ki == NK - 1)
    def _():
        out_ref[...] = acc_s[...]


def _fuse_call(attn, idx, query, memory_values):
    return pl.pallas_call(
        _fuse_body,
        grid=(NQ, NK),
        in_specs=[
            pl.BlockSpec((BQ, TOPK), lambda qi, ki: (qi, 0)),
            pl.BlockSpec((BQ, TOPK), lambda qi, ki: (qi, 0)),
            pl.BlockSpec((BQ, D), lambda qi, ki: (qi, 0)),
            pl.BlockSpec((BK, D), lambda qi, ki: (ki, 0)),
        ],
        out_specs=pl.BlockSpec((BQ, D), lambda qi, ki: (qi, 0)),
        out_shape=jax.ShapeDtypeStruct((Q, D), jnp.float32),
        scratch_shapes=[pltpu.VMEM((BQ, D), jnp.float32)],
        compiler_params=pltpu.CompilerParams(
            dimension_semantics=("parallel", "arbitrary")),
    )(attn, idx, query, memory_values)


NW = 32
QPW = Q // NW
GQ = 4
NG = QPW // GQ
LANES = 16
DC = D // LANES


def _sc_fuse_body(v_hbm, q_hbm, idx_hbm, wb_hbm, out_hbm,
                  idx_v, wb_v, rows_v, acc_v, sem0, sem1):
    wid = lax.axis_index("s") * 2 + lax.axis_index("c")
    qbase = wid * QPW
    sems = (sem0, sem1)
    pltpu.sync_copy(q_hbm.at[pl.ds(qbase, QPW)], acc_v)
    pltpu.sync_copy(wb_hbm.at[pl.ds(qbase * TOPK, QPW * TOPK)], wb_v)
    pltpu.sync_copy(idx_hbm.at[wid * NG], idx_v.at[0])
    pend = pltpu.async_copy(v_hbm.at[idx_v.at[0]], rows_v.at[0], sems[0])
    for g in range(NG):
        buf = g % 2
        if g + 1 < NG:
            nbuf = (g + 1) % 2
            pltpu.sync_copy(idx_hbm.at[wid * NG + g + 1], idx_v.at[nbuf])
            nxt = pltpu.async_copy(v_hbm.at[idx_v.at[nbuf]],
                                   rows_v.at[nbuf], sems[nbuf])
        pend.wait()
        for q in range(GQ):
            qi = g * GQ + q
            wvec = [wb_v[qi * TOPK + j] for j in range(TOPK)]

            def cbody(c, _, q=q, qi=qi, wvec=wvec, buf=buf):
                sl = pl.ds(c * LANES, LANES)
                r = [rows_v[buf, q * TOPK + j, sl] for j in range(TOPK)]
                t01 = wvec[0] * r[0] + wvec[1] * r[1]
                t23 = wvec[2] * r[2] + wvec[3] * r[3]
                t45 = wvec[4] * r[4] + wvec[5] * r[5]
                t67 = wvec[6] * r[6] + wvec[7] * r[7]
                acc_v[qi, sl] = acc_v[qi, sl] + ((t01 + t23) + (t45 + t67))
                return 0

            lax.fori_loop(0, DC, cbody, 0)
        if g + 1 < NG:
            pend = nxt
    pltpu.sync_copy(acc_v, out_hbm.at[pl.ds(qbase, QPW)])


@functools.cache
def _sc_fuse_kernel():
    return functools.partial(
        pl.kernel,
        mesh=plsc.VectorSubcoreMesh(core_axis_name="c",
                                    subcore_axis_name="s"),
        out_type=jax.ShapeDtypeStruct((Q, D), jnp.float32),
        scratch_types=[
            pltpu.VMEM((2, GQ * TOPK), jnp.int32),
            pltpu.VMEM((QPW * TOPK, LANES), jnp.float32),
            pltpu.VMEM((2, GQ * TOPK, D), jnp.float32),
            pltpu.VMEM((QPW, D), jnp.float32),
            pltpu.SemaphoreType.DMA,
            pltpu.SemaphoreType.DMA,
        ],
    )(_sc_fuse_body)


def kernel(query, memory_keys, memory_values, memory_importance):
    imp2d = memory_importance.reshape(1, K)
    attn, idx = _topk_call(query, memory_keys, imp2d)
    wb = jnp.broadcast_to((attn * 0.08)[:, :, None], (Q, TOPK, LANES))
    wb = wb.reshape(Q * TOPK, LANES)
    idx3 = idx.reshape(NW * NG, GQ * TOPK)
    return _sc_fuse_kernel()(memory_values, query, idx3, wb)
